# Initial kernel scaffold; baseline (speedup 1.0000x reference)
import jax
import jax.numpy as jnp
from jax.experimental import pallas as pl


def kernel(x, edge_index, W1, b1, W2, b2):
    raise NotImplementedError("write your pallas kernel here")


# speedup vs baseline: 12.3447x; 12.3447x over previous
import functools

import jax
import jax.numpy as jnp
from jax import lax
from jax.experimental import pallas as pl
from jax.experimental.pallas import tpu as pltpu
from jax.experimental.pallas import tpu_sc as plsc

N = 10000
NP = 10240
E = 320000
D = 128

NC = 2
NS = 16
NW = NC * NS
EPW = E // NW
K = 80
NB = EPW // K
RPT = NP // NS

ROW_BLK = 2000
GRID = N // ROW_BLK

_mesh = plsc.VectorSubcoreMesh(
    core_axis_name="c", subcore_axis_name="s", num_cores=NC, num_subcores=NS
)


@functools.partial(
    pl.kernel,
    out_type=jax.ShapeDtypeStruct((NC, NP, D), jnp.float32),
    mesh=_mesh,
    scratch_types=[
        pltpu.VMEM((K,), jnp.int32),
        pltpu.VMEM((K, D), jnp.float32),
        pltpu.VMEM_SHARED((NP, D), jnp.float32),
    ],
)
def _deg_kernel(dst_hbm, ones_hbm, z_hbm, out_hbm, idx_v, ones_v, deg_sh):
    cid = lax.axis_index("c")
    sid = lax.axis_index("s")
    pltpu.sync_copy(ones_hbm, ones_v)
    pltpu.sync_copy(
        z_hbm.at[pl.ds(sid * RPT, RPT)], deg_sh.at[pl.ds(sid * RPT, RPT)]
    )
    plsc.subcore_barrier()
    base = cid * (E // NC) + sid * EPW

    def step(b, carry):
        pltpu.sync_copy(dst_hbm.at[pl.ds(base + b * K, K)], idx_v)
        pltpu.sync_copy(ones_v, deg_sh.at[idx_v], add=True)
        return carry

    lax.fori_loop(0, NB, step, 0)
    plsc.subcore_barrier()
    pltpu.sync_copy(
        deg_sh.at[pl.ds(sid * RPT, RPT)],
        out_hbm.at[cid, pl.ds(sid * RPT, RPT)],
    )


@functools.partial(
    pl.kernel,
    out_type=jax.ShapeDtypeStruct((NC, NP, D), jnp.float32),
    mesh=_mesh,
    scratch_types=[
        pltpu.VMEM((K,), jnp.int32),
        pltpu.VMEM((K,), jnp.int32),
        pltpu.VMEM((K, D), jnp.float32),
        pltpu.SemaphoreType.DMA,
        pltpu.VMEM_SHARED((NP, D), jnp.float32),
    ],
)
def _agg_kernel(src_hbm, dst_hbm, g_hbm, z_hbm, out_hbm, sidx, didx, rows, sem, acc_sh):
    cid = lax.axis_index("c")
    sid = lax.axis_index("s")
    pltpu.sync_copy(
        z_hbm.at[pl.ds(sid * RPT, RPT)], acc_sh.at[pl.ds(sid * RPT, RPT)]
    )
    plsc.subcore_barrier()
    base = cid * (E // NC) + sid * EPW

    def step(b, carry):
        pltpu.sync_copy(src_hbm.at[pl.ds(base + b * K, K)], sidx)
        pltpu.sync_copy(dst_hbm.at[pl.ds(base + b * K, K)], didx)
        pltpu.async_copy(g_hbm.at[sidx], rows, sem).wait()
        pltpu.sync_copy(rows, acc_sh.at[didx], add=True)
        return carry

    lax.fori_loop(0, NB, step, 0)
    plsc.subcore_barrier()
    pltpu.sync_copy(
        acc_sh.at[pl.ds(sid * RPT, RPT)],
        out_hbm.at[cid, pl.ds(sid * RPT, RPT)],
    )


def _dis(deg_ref):
    deg = deg_ref[0, :, :1] + deg_ref[1, :, :1] + 1.0
    return lax.rsqrt(deg)


def _k1_body(x_ref, w_ref, deg_ref, g_ref):
    h = jnp.dot(x_ref[...], w_ref[...], preferred_element_type=jnp.float32)
    g_ref[...] = _dis(deg_ref) * h


def _k2_body(acc_ref, g1_ref, deg_ref, w_ref, b_ref, g2_ref):
    dis = _dis(deg_ref)
    t = dis * (acc_ref[0] + acc_ref[1] + g1_ref[...]) + b_ref[...]
    z = jnp.maximum(t, 0.0)
    h = jnp.dot(z, w_ref[...], preferred_element_type=jnp.float32)
    g2_ref[...] = dis * h


def _k3_body(acc_ref, g2_ref, deg_ref, b_ref, out_ref):
    dis = _dis(deg_ref)
    out_ref[...] = dis * (acc_ref[0] + acc_ref[1] + g2_ref[...]) + b_ref[...]


_row_spec = pl.BlockSpec((ROW_BLK, D), lambda i: (i, 0))
_acc_spec = pl.BlockSpec((NC, ROW_BLK, D), lambda i: (0, i, 0))
_deg_spec = pl.BlockSpec((NC, ROW_BLK, D), lambda i: (0, i, 0))
_w_spec = pl.BlockSpec((D, D), lambda i: (0, 0))
_b_spec = pl.BlockSpec((1, D), lambda i: (0, 0))
_out_struct = jax.ShapeDtypeStruct((N, D), jnp.float32)

_k1 = pl.pallas_call(
    _k1_body,
    grid=(GRID,),
    in_specs=[_row_spec, _w_spec, _deg_spec],
    out_specs=_row_spec,
    out_shape=_out_struct,
)

_k2 = pl.pallas_call(
    _k2_body,
    grid=(GRID,),
    in_specs=[_acc_spec, _row_spec, _deg_spec, _w_spec, _b_spec],
    out_specs=_row_spec,
    out_shape=_out_struct,
)

_k3 = pl.pallas_call(
    _k3_body,
    grid=(GRID,),
    in_specs=[_acc_spec, _row_spec, _deg_spec, _b_spec],
    out_specs=_row_spec,
    out_shape=_out_struct,
)


def kernel(x, edge_index, W1, b1, W2, b2):
    src = edge_index[0].astype(jnp.int32)
    dst = edge_index[1].astype(jnp.int32)
    ones128 = jnp.ones((K, D), jnp.float32)
    z128 = jnp.zeros((NP, D), jnp.float32)
    b1r = b1.reshape(1, D)
    b2r = b2.reshape(1, D)

    deg = _deg_kernel(dst, ones128, z128)
    g1 = _k1(x, W1, deg)
    acc1 = _agg_kernel(src, dst, g1, z128)
    g2 = _k2(acc1, g1, deg, W2, b1r)
    acc2 = _agg_kernel(src, dst, g2, z128)
    return _k3(acc2, g2, deg, b2r)


# speedup vs baseline: 14.0813x; 1.1407x over previous
import functools

import jax
import jax.numpy as jnp
from jax import lax
from jax.experimental import pallas as pl
from jax.experimental.pallas import tpu as pltpu
from jax.experimental.pallas import tpu_sc as plsc

N = 10000
NP = 10240
E = 320000
D = 128

NC = 2
NS = 16
NW = NC * NS
K = 128
NB = 79
EPW = NB * K
EP = NW * EPW
RPT = NP // NS

ROW_BLK = 2000
GRID = N // ROW_BLK

_mesh = plsc.VectorSubcoreMesh(
    core_axis_name="c", subcore_axis_name="s", num_cores=NC, num_subcores=NS
)


@functools.partial(
    pl.kernel,
    out_type=jax.ShapeDtypeStruct((NC, NP, D), jnp.float32),
    mesh=_mesh,
    scratch_types=[
        pltpu.VMEM((NB, K), jnp.int32),
        pltpu.VMEM((K, D), jnp.float32),
        pltpu.SemaphoreType.DMA,
        pltpu.VMEM_SHARED((NP, D), jnp.float32),
    ],
)
def _deg_kernel(dst_hbm, ones_hbm, z_hbm, out_hbm, dstage, ones_v, sem, deg_sh):
    cid = lax.axis_index("c")
    sid = lax.axis_index("s")
    w = cid * NS + sid
    pltpu.sync_copy(ones_hbm, ones_v)
    pltpu.sync_copy(dst_hbm.at[w], dstage)
    pltpu.sync_copy(
        z_hbm.at[pl.ds(sid * RPT, RPT)], deg_sh.at[pl.ds(sid * RPT, RPT)]
    )
    plsc.subcore_barrier()

    def fire(b, carry):
        pltpu.async_copy(ones_v, deg_sh.at[dstage.at[b]], sem, add=True)
        return carry

    lax.fori_loop(0, NB, fire, 0)

    def drain(b, carry):
        pltpu.make_async_copy(z_hbm.at[pl.ds(0, K)], ones_v, sem).wait()
        return carry

    lax.fori_loop(0, NB, drain, 0)
    plsc.subcore_barrier()
    pltpu.sync_copy(
        deg_sh.at[pl.ds(sid * RPT, RPT)],
        out_hbm.at[cid, pl.ds(sid * RPT, RPT)],
    )


@functools.partial(
    pl.kernel,
    out_type=jax.ShapeDtypeStruct((NC, NP, D), jnp.float32),
    mesh=_mesh,
    scratch_types=[
        pltpu.VMEM((2, K), jnp.int32),
        pltpu.VMEM((NB, K), jnp.int32),
        pltpu.VMEM((2, K, D), jnp.float32),
        pltpu.SemaphoreType.DMA((2,)),
        pltpu.SemaphoreType.DMA((2,)),
        pltpu.VMEM_SHARED((NP, D), jnp.float32),
    ],
)
def _agg_kernel(
    src_hbm, dst_hbm, g_hbm, z_hbm, out_hbm,
    sring, dstage, rows, gsem, isem, acc_sh,
):
    cid = lax.axis_index("c")
    sid = lax.axis_index("s")
    w = cid * NS + sid
    ebase = w * EPW
    pltpu.sync_copy(dst_hbm.at[w], dstage)
    pltpu.sync_copy(
        z_hbm.at[pl.ds(sid * RPT, RPT)], acc_sh.at[pl.ds(sid * RPT, RPT)]
    )
    plsc.subcore_barrier()

    pltpu.sync_copy(src_hbm.at[pl.ds(ebase, K)], sring.at[0])
    pltpu.async_copy(g_hbm.at[sring.at[0]], rows.at[0], gsem.at[0])
    pltpu.async_copy(src_hbm.at[pl.ds(ebase + K, K)], sring.at[1], isem.at[1])

    def body(b, carry):
        p = lax.rem(b, 2)
        q = 1 - p

        @pl.when(b + 1 < NB)
        def _():
            pltpu.make_async_copy(
                src_hbm.at[pl.ds(ebase, K)], sring.at[q], isem.at[q]
            ).wait()
            pltpu.async_copy(g_hbm.at[sring.at[q]], rows.at[q], gsem.at[q])

        pltpu.make_async_copy(
            z_hbm.at[pl.ds(0, K)], rows.at[p], gsem.at[p]
        ).wait()

        @pl.when(b + 2 < NB)
        def _():
            pltpu.async_copy(
                src_hbm.at[pl.ds(ebase + (b + 2) * K, K)],
                sring.at[p],
                isem.at[p],
            )

        pltpu.sync_copy(rows.at[p], acc_sh.at[dstage.at[b]], add=True)
        return carry

    lax.fori_loop(0, NB, body, 0)

    plsc.subcore_barrier()
    pltpu.sync_copy(
        acc_sh.at[pl.ds(sid * RPT, RPT)],
        out_hbm.at[cid, pl.ds(sid * RPT, RPT)],
    )


def _dis(deg_ref):
    deg = deg_ref[0, :, :1] + deg_ref[1, :, :1] + 1.0
    return lax.rsqrt(deg)


def _k1_body(x_ref, w_ref, deg_ref, g_ref):
    h = jnp.dot(x_ref[...], w_ref[...], preferred_element_type=jnp.float32)
    g_ref[...] = _dis(deg_ref) * h


def _k2_body(acc_ref, g1_ref, deg_ref, w_ref, b_ref, g2_ref):
    dis = _dis(deg_ref)
    t = dis * (acc_ref[0] + acc_ref[1] + g1_ref[...]) + b_ref[...]
    z = jnp.maximum(t, 0.0)
    h = jnp.dot(z, w_ref[...], preferred_element_type=jnp.float32)
    g2_ref[...] = dis * h


def _k3_body(acc_ref, g2_ref, deg_ref, b_ref, out_ref):
    dis = _dis(deg_ref)
    out_ref[...] = dis * (acc_ref[0] + acc_ref[1] + g2_ref[...]) + b_ref[...]


_row_spec = pl.BlockSpec((ROW_BLK, D), lambda i: (i, 0))
_acc_spec = pl.BlockSpec((NC, ROW_BLK, D), lambda i: (0, i, 0))
_deg_spec = pl.BlockSpec((NC, ROW_BLK, D), lambda i: (0, i, 0))
_w_spec = pl.BlockSpec((D, D), lambda i: (0, 0))
_b_spec = pl.BlockSpec((1, D), lambda i: (0, 0))
_out_struct = jax.ShapeDtypeStruct((N, D), jnp.float32)

_k1 = pl.pallas_call(
    _k1_body,
    grid=(GRID,),
    in_specs=[_row_spec, _w_spec, _deg_spec],
    out_specs=_row_spec,
    out_shape=_out_struct,
)

_k2 = pl.pallas_call(
    _k2_body,
    grid=(GRID,),
    in_specs=[_acc_spec, _row_spec, _deg_spec, _w_spec, _b_spec],
    out_specs=_row_spec,
    out_shape=_out_struct,
)

_k3 = pl.pallas_call(
    _k3_body,
    grid=(GRID,),
    in_specs=[_acc_spec, _row_spec, _deg_spec, _b_spec],
    out_specs=_row_spec,
    out_shape=_out_struct,
)


def kernel(x, edge_index, W1, b1, W2, b2):
    e32 = edge_index.astype(jnp.int32)
    src = jnp.concatenate([e32[0], jnp.zeros((EP - E,), jnp.int32)])
    dst = jnp.concatenate(
        [e32[1], jnp.full((EP - E,), NP - 1, jnp.int32)]
    ).reshape(NW, NB, K)
    ones128 = jnp.ones((K, D), jnp.float32)
    z128 = jnp.zeros((NP, D), jnp.float32)
    b1r = b1.reshape(1, D)
    b2r = b2.reshape(1, D)

    deg = _deg_kernel(dst, ones128, z128)
    g1 = _k1(x, W1, deg)
    acc1 = _agg_kernel(src, dst, g1, z128)
    g2 = _k2(acc1, g1, deg, W2, b1r)
    acc2 = _agg_kernel(src, dst, g2, z128)
    return _k3(acc2, g2, deg, b2r)


# speedup vs baseline: 28.3066x; 2.0102x over previous
import functools

import jax
import jax.numpy as jnp
from jax import lax
from jax.experimental import pallas as pl
from jax.experimental.pallas import tpu as pltpu
from jax.experimental.pallas import tpu_sc as plsc

N = 10000
NP = 10240
E = 320000
D = 128

NC = 2
NS = 16
NW = NC * NS
K = 128
NB = 79
EPW = NB * K
EP = NW * EPW
RPT = NP // NS

ROW_BLK = 2000
GRID = N // ROW_BLK

_mesh = plsc.VectorSubcoreMesh(
    core_axis_name="c", subcore_axis_name="s", num_cores=NC, num_subcores=NS
)


@functools.partial(
    pl.kernel,
    out_type=jax.ShapeDtypeStruct((NC, NP, D), jnp.float32),
    mesh=_mesh,
    scratch_types=[
        pltpu.VMEM((NB, K), jnp.int32),
        pltpu.VMEM((K, D), jnp.float32),
        pltpu.SemaphoreType.DMA,
        pltpu.VMEM_SHARED((NP, D), jnp.float32),
    ],
)
def _deg_kernel(dst_hbm, ones_hbm, z_hbm, out_hbm, dstage, ones_v, sem, deg_sh):
    cid = lax.axis_index("c")
    sid = lax.axis_index("s")
    w = cid * NS + sid
    pltpu.sync_copy(ones_hbm, ones_v)
    pltpu.sync_copy(dst_hbm.at[w], dstage)
    pltpu.sync_copy(
        z_hbm.at[pl.ds(sid * RPT, RPT)], deg_sh.at[pl.ds(sid * RPT, RPT)]
    )
    plsc.subcore_barrier()

    def fire(b, carry):
        pltpu.async_copy(ones_v, deg_sh.at[dstage.at[b]], sem, add=True)
        return carry

    lax.fori_loop(0, NB, fire, 0)

    def drain(b, carry):
        pltpu.make_async_copy(z_hbm.at[pl.ds(0, K)], ones_v, sem).wait()
        return carry

    lax.fori_loop(0, NB, drain, 0)
    plsc.subcore_barrier()
    pltpu.sync_copy(
        deg_sh.at[pl.ds(sid * RPT, RPT)],
        out_hbm.at[cid, pl.ds(sid * RPT, RPT)],
    )


@functools.partial(
    pl.kernel,
    out_type=jax.ShapeDtypeStruct((NC, NP, D), jnp.float32),
    mesh=_mesh,
    scratch_types=[
        pltpu.VMEM((2, K), jnp.int32),
        pltpu.VMEM((NB, K), jnp.int32),
        pltpu.VMEM((2, K, D), jnp.float32),
        pltpu.SemaphoreType.DMA((2,)),
        pltpu.SemaphoreType.DMA((2,)),
        pltpu.VMEM_SHARED((NP, D), jnp.float32),
    ],
)
def _agg_kernel(
    src_hbm, dst_hbm, g_hbm, z_hbm, out_hbm,
    sring, dstage, rows, gsem, isem, acc_sh,
):
    cid = lax.axis_index("c")
    sid = lax.axis_index("s")
    w = cid * NS + sid
    ebase = w * EPW
    pltpu.sync_copy(dst_hbm.at[w], dstage)
    pltpu.sync_copy(
        z_hbm.at[pl.ds(sid * RPT, RPT)], acc_sh.at[pl.ds(sid * RPT, RPT)]
    )
    plsc.subcore_barrier()

    pltpu.sync_copy(src_hbm.at[pl.ds(ebase, K)], sring.at[0])
    pltpu.async_copy(g_hbm.at[sring.at[0]], rows.at[0], gsem.at[0])
    pltpu.async_copy(src_hbm.at[pl.ds(ebase + K, K)], sring.at[1], isem.at[1])

    def body(b, carry):
        p = lax.rem(b, 2)
        q = 1 - p

        @pl.when(b + 1 < NB)
        def _():
            pltpu.make_async_copy(
                src_hbm.at[pl.ds(ebase, K)], sring.at[q], isem.at[q]
            ).wait()
            pltpu.async_copy(g_hbm.at[sring.at[q]], rows.at[q], gsem.at[q])

        pltpu.make_async_copy(
            z_hbm.at[pl.ds(0, K)], rows.at[p], gsem.at[p]
        ).wait()

        @pl.when(b + 2 < NB)
        def _():
            pltpu.async_copy(
                src_hbm.at[pl.ds(ebase + (b + 2) * K, K)],
                sring.at[p],
                isem.at[p],
            )

        pltpu.sync_copy(rows.at[p], acc_sh.at[dstage.at[b]], add=True)
        return carry

    lax.fori_loop(0, NB, body, 0)

    plsc.subcore_barrier()
    pltpu.sync_copy(
        acc_sh.at[pl.ds(sid * RPT, RPT)],
        out_hbm.at[cid, pl.ds(sid * RPT, RPT)],
    )


def _dis(deg_ref):
    deg = deg_ref[0, :, :1] + deg_ref[1, :, :1] + 1.0
    return lax.rsqrt(deg)


def _k1_body(x_ref, w_ref, deg_ref, g_ref):
    h = jnp.dot(x_ref[...], w_ref[...], preferred_element_type=jnp.float32)
    g_ref[...] = _dis(deg_ref) * h


def _k2_body(acc_ref, g1_ref, deg_ref, w_ref, b_ref, g2_ref):
    dis = _dis(deg_ref)
    t = dis * (acc_ref[0] + acc_ref[1] + g1_ref[...]) + b_ref[...]
    z = jnp.maximum(t, 0.0)
    h = jnp.dot(z, w_ref[...], preferred_element_type=jnp.float32)
    g2_ref[...] = dis * h


def _k3_body(acc_ref, g2_ref, deg_ref, b_ref, out_ref):
    dis = _dis(deg_ref)
    out_ref[...] = dis * (acc_ref[0] + acc_ref[1] + g2_ref[...]) + b_ref[...]


_row_spec = pl.BlockSpec((ROW_BLK, D), lambda i: (i, 0))
_acc_spec = pl.BlockSpec((NC, ROW_BLK, D), lambda i: (0, i, 0))
_deg_spec = pl.BlockSpec((NC, ROW_BLK, D), lambda i: (0, i, 0))
_w_spec = pl.BlockSpec((D, D), lambda i: (0, 0))
_b_spec = pl.BlockSpec((1, D), lambda i: (0, 0))
_out_struct = jax.ShapeDtypeStruct((N, D), jnp.float32)

_k1 = pl.pallas_call(
    _k1_body,
    grid=(GRID,),
    in_specs=[_row_spec, _w_spec, _deg_spec],
    out_specs=_row_spec,
    out_shape=_out_struct,
)

_k2 = pl.pallas_call(
    _k2_body,
    grid=(GRID,),
    in_specs=[_acc_spec, _row_spec, _deg_spec, _w_spec, _b_spec],
    out_specs=_row_spec,
    out_shape=_out_struct,
)

_k3 = pl.pallas_call(
    _k3_body,
    grid=(GRID,),
    in_specs=[_acc_spec, _row_spec, _deg_spec, _b_spec],
    out_specs=_row_spec,
    out_shape=_out_struct,
)


def kernel(x, edge_index, W1, b1, W2, b2):
    e32 = edge_index.astype(jnp.int32)
    pad = jnp.arange(EP - E, dtype=jnp.int32)
    src = jnp.concatenate([e32[0], pad % N])
    dst = jnp.concatenate([e32[1], N + pad % (NP - N)]).reshape(NW, NB, K)
    ones128 = jnp.ones((K, D), jnp.float32)
    z128 = jnp.zeros((NP, D), jnp.float32)
    b1r = b1.reshape(1, D)
    b2r = b2.reshape(1, D)

    deg = _deg_kernel(dst, ones128, z128)
    g1 = _k1(x, W1, deg)
    acc1 = _agg_kernel(src, dst, g1, z128)
    g2 = _k2(acc1, g1, deg, W2, b1r)
    acc2 = _agg_kernel(src, dst, g2, z128)
    return _k3(acc2, g2, deg, b2r)


# speedup vs baseline: 33.9416x; 1.1991x over previous
import functools

import jax
import jax.numpy as jnp
from jax import lax
from jax.experimental import pallas as pl
from jax.experimental.pallas import tpu as pltpu
from jax.experimental.pallas import tpu_sc as plsc

N = 10000
NP = 10240
E = 320000
D = 128

NC = 2
NS = 16
NW = NC * NS
K = 128
NB = 79
EPW = NB * K
EP = NW * EPW
RPT = NP // NS

ROW_BLK = 2048
GRID = NP // ROW_BLK

_mesh = plsc.VectorSubcoreMesh(
    core_axis_name="c", subcore_axis_name="s", num_cores=NC, num_subcores=NS
)


@functools.partial(
    pl.kernel,
    out_type=jax.ShapeDtypeStruct((NW, 8, NP), jnp.int32),
    mesh=_mesh,
    scratch_types=[
        pltpu.VMEM((EPW,), jnp.int32),
        pltpu.VMEM((NP,), jnp.int32),
    ],
    compiler_params=pltpu.CompilerParams(needs_layout_passes=False),
)
def _deg_kernel(dst_hbm, out_hbm, dstage, hist):
    cid = lax.axis_index("c")
    sid = lax.axis_index("s")
    w = cid * NS + sid
    pltpu.sync_copy(dst_hbm.at[pl.ds(w * EPW, EPW)], dstage)

    def zero(i, carry):
        hist[pl.ds(i * 16, 16)] = jnp.zeros((16,), jnp.int32)
        return carry

    lax.fori_loop(0, NP // 16, zero, 0)

    ones_v = jnp.ones((16,), jnp.int32)

    def step(t, carry):
        idx = dstage[pl.ds(t * 16, 16)]
        plsc.addupdate_scatter(hist, [idx], ones_v)
        return carry

    lax.fori_loop(0, EPW // 16, step, 0)
    pltpu.sync_copy(hist, out_hbm.at[w, 0])


@functools.partial(
    pl.kernel,
    out_type=jax.ShapeDtypeStruct((NC, NP, D), jnp.float32),
    mesh=_mesh,
    scratch_types=[
        pltpu.VMEM((2, K), jnp.int32),
        pltpu.VMEM((NB, K), jnp.int32),
        pltpu.VMEM((2, K, D), jnp.float32),
        pltpu.SemaphoreType.DMA((2,)),
        pltpu.SemaphoreType.DMA((2,)),
        pltpu.VMEM_SHARED((NP, D), jnp.float32),
    ],
)
def _agg_kernel(
    src_hbm, dst_hbm, g_hbm, z_hbm, out_hbm,
    sring, dstage, rows, gsem, isem, acc_sh,
):
    cid = lax.axis_index("c")
    sid = lax.axis_index("s")
    w = cid * NS + sid
    ebase = w * EPW
    pltpu.sync_copy(dst_hbm.at[w], dstage)
    pltpu.sync_copy(
        z_hbm.at[pl.ds(sid * RPT, RPT)], acc_sh.at[pl.ds(sid * RPT, RPT)]
    )
    plsc.subcore_barrier()

    pltpu.sync_copy(src_hbm.at[pl.ds(ebase, K)], sring.at[0])
    pltpu.async_copy(g_hbm.at[sring.at[0]], rows.at[0], gsem.at[0])
    pltpu.async_copy(src_hbm.at[pl.ds(ebase + K, K)], sring.at[1], isem.at[1])

    def body(b, carry):
        p = lax.rem(b, 2)
        q = 1 - p

        @pl.when(b + 1 < NB)
        def _():
            pltpu.make_async_copy(
                src_hbm.at[pl.ds(ebase, K)], sring.at[q], isem.at[q]
            ).wait()
            pltpu.async_copy(g_hbm.at[sring.at[q]], rows.at[q], gsem.at[q])

        pltpu.make_async_copy(
            z_hbm.at[pl.ds(0, K)], rows.at[p], gsem.at[p]
        ).wait()

        @pl.when(b + 2 < NB)
        def _():
            pltpu.async_copy(
                src_hbm.at[pl.ds(ebase + (b + 2) * K, K)],
                sring.at[p],
                isem.at[p],
            )

        pltpu.sync_copy(rows.at[p], acc_sh.at[dstage.at[b]], add=True)
        return carry

    lax.fori_loop(0, NB, body, 0)

    plsc.subcore_barrier()
    pltpu.sync_copy(
        acc_sh.at[pl.ds(sid * RPT, RPT)],
        out_hbm.at[cid, pl.ds(sid * RPT, RPT)],
    )


def _k1_body(x_ref, w_ref, deg_ref, g_ref, dis_ref):
    degp = deg_ref[:, 0, :].astype(jnp.float32)
    deg_col = jnp.dot(
        degp.T, jnp.ones((NW, 1), jnp.float32),
        preferred_element_type=jnp.float32,
    ) + 1.0
    dis = lax.rsqrt(deg_col)
    dis_ref[...] = jnp.broadcast_to(dis, (ROW_BLK, D))
    h = jnp.dot(x_ref[...], w_ref[...], preferred_element_type=jnp.float32)
    g_ref[...] = dis * h


def _k2_body(acc_ref, g1_ref, dis_ref, w_ref, b_ref, g2_ref):
    dis = dis_ref[...]
    t = dis * (acc_ref[0] + acc_ref[1] + g1_ref[...]) + b_ref[...]
    z = jnp.maximum(t, 0.0)
    h = jnp.dot(z, w_ref[...], preferred_element_type=jnp.float32)
    g2_ref[...] = dis * h


def _k3_body(acc_ref, g2_ref, dis_ref, b_ref, out_ref):
    out_ref[...] = (
        dis_ref[...] * (acc_ref[0] + acc_ref[1] + g2_ref[...]) + b_ref[...]
    )


_row_spec = pl.BlockSpec((ROW_BLK, D), lambda i: (i, 0))
_acc_spec = pl.BlockSpec((NC, ROW_BLK, D), lambda i: (0, i, 0))
_deg_spec = pl.BlockSpec((NW, 8, ROW_BLK), lambda i: (0, 0, i))
_w_spec = pl.BlockSpec((D, D), lambda i: (0, 0))
_b_spec = pl.BlockSpec((1, D), lambda i: (0, 0))
_out_struct = jax.ShapeDtypeStruct((NP, D), jnp.float32)

_k1 = pl.pallas_call(
    _k1_body,
    grid=(GRID,),
    in_specs=[_row_spec, _w_spec, _deg_spec],
    out_specs=[_row_spec, _row_spec],
    out_shape=[_out_struct, _out_struct],
)

_k2 = pl.pallas_call(
    _k2_body,
    grid=(GRID,),
    in_specs=[_acc_spec, _row_spec, _row_spec, _w_spec, _b_spec],
    out_specs=_row_spec,
    out_shape=_out_struct,
)

_k3 = pl.pallas_call(
    _k3_body,
    grid=(GRID,),
    in_specs=[_acc_spec, _row_spec, _row_spec, _b_spec],
    out_specs=_row_spec,
    out_shape=_out_struct,
)


def kernel(x, edge_index, W1, b1, W2, b2):
    e32 = edge_index.astype(jnp.int32)
    pad = jnp.arange(EP - E, dtype=jnp.int32)
    src = jnp.concatenate([e32[0], pad % N])
    dst1 = jnp.concatenate([e32[1], N + pad % (NP - N)])
    dst = dst1.reshape(NW, NB, K)
    z128 = jnp.zeros((NP, D), jnp.float32)
    xp = jnp.concatenate([x, jnp.zeros((NP - N, D), x.dtype)])
    b1r = b1.reshape(1, D)
    b2r = b2.reshape(1, D)

    deg = _deg_kernel(dst1)
    g1, dis = _k1(xp, W1, deg)
    acc1 = _agg_kernel(src, dst, g1, z128)
    g2 = _k2(acc1, g1, dis, W2, b1r)
    acc2 = _agg_kernel(src, dst, g2, z128)
    return _k3(acc2, g2, dis, b2r)[:N]


# speedup vs baseline: 34.0201x; 1.0023x over previous
import functools

import jax
import jax.numpy as jnp
from jax import lax
from jax.experimental import pallas as pl
from jax.experimental.pallas import tpu as pltpu
from jax.experimental.pallas import tpu_sc as plsc

N = 10000
NP = 10240
E = 320000
D = 128

NC = 2
NS = 16
NW = NC * NS
K = 128
NB = 79
EPW = NB * K
EP = NW * EPW
RPT = NP // NS

ROW_BLK = 2048
GRID = NP // ROW_BLK

_mesh = plsc.VectorSubcoreMesh(
    core_axis_name="c", subcore_axis_name="s", num_cores=NC, num_subcores=NS
)


@functools.partial(
    pl.kernel,
    out_type=jax.ShapeDtypeStruct((NW, 8, NP), jnp.int32),
    mesh=_mesh,
    scratch_types=[
        pltpu.VMEM((EPW,), jnp.int32),
        pltpu.VMEM((NP,), jnp.int32),
    ],
    compiler_params=pltpu.CompilerParams(needs_layout_passes=False),
)
def _deg_kernel(dst_hbm, out_hbm, dstage, hist):
    cid = lax.axis_index("c")
    sid = lax.axis_index("s")
    w = cid * NS + sid
    pltpu.sync_copy(dst_hbm.at[pl.ds(w * EPW, EPW)], dstage)

    def zero(i, carry):
        hist[pl.ds(i * 16, 16)] = jnp.zeros((16,), jnp.int32)
        return carry

    lax.fori_loop(0, NP // 16, zero, 0)

    ones_v = jnp.ones((16,), jnp.int32)

    def step(t, carry):
        idx = dstage[pl.ds(t * 16, 16)]
        plsc.addupdate_scatter(hist, [idx], ones_v)
        return carry

    lax.fori_loop(0, EPW // 16, step, 0)
    pltpu.sync_copy(hist, out_hbm.at[w, 0])


@functools.partial(
    pl.kernel,
    out_type=jax.ShapeDtypeStruct((NC, NP, D), jnp.float32),
    mesh=_mesh,
    scratch_types=[
        pltpu.VMEM((2, K), jnp.int32),
        pltpu.VMEM((NB, K), jnp.int32),
        pltpu.VMEM((2, K, D), jnp.float32),
        pltpu.SemaphoreType.DMA((2,)),
        pltpu.SemaphoreType.DMA((2,)),
        pltpu.SemaphoreType.DMA((2,)),
        pltpu.VMEM_SHARED((NP, D), jnp.float32),
    ],
)
def _agg_kernel(
    src_hbm, dst_hbm, g_hbm, z_hbm, out_hbm,
    sring, dstage, rows, gsem, isem, ssem, acc_sh,
):
    cid = lax.axis_index("c")
    sid = lax.axis_index("s")
    w = cid * NS + sid
    ebase = w * EPW
    pltpu.sync_copy(dst_hbm.at[w], dstage)
    pltpu.sync_copy(
        z_hbm.at[pl.ds(sid * RPT, RPT)], acc_sh.at[pl.ds(sid * RPT, RPT)]
    )
    plsc.subcore_barrier()

    pltpu.sync_copy(src_hbm.at[pl.ds(ebase, K)], sring.at[0])
    pltpu.async_copy(g_hbm.at[sring.at[0]], rows.at[0], gsem.at[0])
    pltpu.async_copy(src_hbm.at[pl.ds(ebase + K, K)], sring.at[1], isem.at[1])

    def body(b, carry):
        p = lax.rem(b, 2)
        q = 1 - p

        @pl.when(b + 1 < NB)
        def _():
            pltpu.make_async_copy(
                src_hbm.at[pl.ds(ebase, K)], sring.at[q], isem.at[q]
            ).wait()

            @pl.when(b >= 1)
            def _():
                pltpu.make_async_copy(
                    z_hbm.at[pl.ds(0, K)], rows.at[q], ssem.at[q]
                ).wait()

            pltpu.async_copy(g_hbm.at[sring.at[q]], rows.at[q], gsem.at[q])

        pltpu.make_async_copy(
            z_hbm.at[pl.ds(0, K)], rows.at[p], gsem.at[p]
        ).wait()

        @pl.when(b + 2 < NB)
        def _():
            pltpu.async_copy(
                src_hbm.at[pl.ds(ebase + (b + 2) * K, K)],
                sring.at[p],
                isem.at[p],
            )

        pltpu.async_copy(rows.at[p], acc_sh.at[dstage.at[b]], ssem.at[p], add=True)
        return carry

    lax.fori_loop(0, NB, body, 0)

    pltpu.make_async_copy(z_hbm.at[pl.ds(0, K)], rows.at[0], ssem.at[0]).wait()
    pltpu.make_async_copy(z_hbm.at[pl.ds(0, K)], rows.at[1], ssem.at[1]).wait()

    plsc.subcore_barrier()
    pltpu.sync_copy(
        acc_sh.at[pl.ds(sid * RPT, RPT)],
        out_hbm.at[cid, pl.ds(sid * RPT, RPT)],
    )


def _k1_body(x_ref, w_ref, deg_ref, g_ref, dis_ref):
    degp = deg_ref[:, 0, :].astype(jnp.float32)
    deg_col = jnp.dot(
        degp.T, jnp.ones((NW, 1), jnp.float32),
        preferred_element_type=jnp.float32,
    ) + 1.0
    dis = lax.rsqrt(deg_col)
    dis_ref[...] = jnp.broadcast_to(dis, (ROW_BLK, D))
    h = jnp.dot(x_ref[...], w_ref[...], preferred_element_type=jnp.float32)
    g_ref[...] = dis * h


def _k2_body(acc_ref, g1_ref, dis_ref, w_ref, b_ref, g2_ref):
    dis = dis_ref[...]
    t = dis * (acc_ref[0] + acc_ref[1] + g1_ref[...]) + b_ref[...]
    z = jnp.maximum(t, 0.0)
    h = jnp.dot(z, w_ref[...], preferred_element_type=jnp.float32)
    g2_ref[...] = dis * h


def _k3_body(acc_ref, g2_ref, dis_ref, b_ref, out_ref):
    out_ref[...] = (
        dis_ref[...] * (acc_ref[0] + acc_ref[1] + g2_ref[...]) + b_ref[...]
    )


_row_spec = pl.BlockSpec((ROW_BLK, D), lambda i: (i, 0))
_acc_spec = pl.BlockSpec((NC, ROW_BLK, D), lambda i: (0, i, 0))
_deg_spec = pl.BlockSpec((NW, 8, ROW_BLK), lambda i: (0, 0, i))
_w_spec = pl.BlockSpec((D, D), lambda i: (0, 0))
_b_spec = pl.BlockSpec((1, D), lambda i: (0, 0))
_out_struct = jax.ShapeDtypeStruct((NP, D), jnp.float32)

_k1 = pl.pallas_call(
    _k1_body,
    grid=(GRID,),
    in_specs=[_row_spec, _w_spec, _deg_spec],
    out_specs=[_row_spec, _row_spec],
    out_shape=[_out_struct, _out_struct],
)

_k2 = pl.pallas_call(
    _k2_body,
    grid=(GRID,),
    in_specs=[_acc_spec, _row_spec, _row_spec, _w_spec, _b_spec],
    out_specs=_row_spec,
    out_shape=_out_struct,
)

_k3 = pl.pallas_call(
    _k3_body,
    grid=(GRID,),
    in_specs=[_acc_spec, _row_spec, _row_spec, _b_spec],
    out_specs=_row_spec,
    out_shape=_out_struct,
)


def kernel(x, edge_index, W1, b1, W2, b2):
    e32 = edge_index.astype(jnp.int32)
    pad = jnp.arange(EP - E, dtype=jnp.int32)
    src = jnp.concatenate([e32[0], pad % N])
    dst1 = jnp.concatenate([e32[1], N + pad % (NP - N)])
    dst = dst1.reshape(NW, NB, K)
    z128 = jnp.zeros((NP, D), jnp.float32)
    xp = jnp.concatenate([x, jnp.zeros((NP - N, D), x.dtype)])
    b1r = b1.reshape(1, D)
    b2r = b2.reshape(1, D)

    deg = _deg_kernel(dst1)
    g1, dis = _k1(xp, W1, deg)
    acc1 = _agg_kernel(src, dst, g1, z128)
    g2 = _k2(acc1, g1, dis, W2, b1r)
    acc2 = _agg_kernel(src, dst, g2, z128)
    return _k3(acc2, g2, dis, b2r)[:N]


# speedup vs baseline: 34.6639x; 1.0189x over previous
import functools

import jax
import jax.numpy as jnp
from jax import lax
from jax.experimental import pallas as pl
from jax.experimental.pallas import tpu as pltpu
from jax.experimental.pallas import tpu_sc as plsc

N = 10000
NP = 10240
E = 320000
D = 128

NC = 2
NS = 16
NW = NC * NS
K = 128
NB = 79
EPW = NB * K
EP = NW * EPW
RPT = NP // NS

ROW_BLK = 2048
GRID = NP // ROW_BLK

_mesh = plsc.VectorSubcoreMesh(
    core_axis_name="c", subcore_axis_name="s", num_cores=NC, num_subcores=NS
)


@functools.partial(
    pl.kernel,
    out_type=jax.ShapeDtypeStruct((NW, 8, NP), jnp.int32),
    mesh=_mesh,
    scratch_types=[
        pltpu.VMEM((EPW,), jnp.int32),
        pltpu.VMEM((NP,), jnp.int32),
    ],
    compiler_params=pltpu.CompilerParams(needs_layout_passes=False),
)
def _deg_kernel(dst_hbm, out_hbm, dstage, hist):
    cid = lax.axis_index("c")
    sid = lax.axis_index("s")
    w = cid * NS + sid
    pltpu.sync_copy(dst_hbm.at[pl.ds(w * EPW, EPW)], dstage)

    def zero(i, carry):
        hist[pl.ds(i * 16, 16)] = jnp.zeros((16,), jnp.int32)
        return carry

    lax.fori_loop(0, NP // 16, zero, 0)

    ones_v = jnp.ones((16,), jnp.int32)

    def step(t, carry):
        idx = dstage[pl.ds(t * 16, 16)]
        plsc.addupdate_scatter(hist, [idx], ones_v)
        return carry

    lax.fori_loop(0, EPW // 16, step, 0)
    pltpu.sync_copy(hist, out_hbm.at[w, 0])


@functools.partial(
    pl.kernel,
    out_type=jax.ShapeDtypeStruct((NC, NP, D), jnp.float32),
    mesh=_mesh,
    scratch_types=[
        pltpu.VMEM((2, K), jnp.int32),
        pltpu.VMEM((NB, K), jnp.int32),
        pltpu.VMEM((2, K, D), jnp.float32),
        pltpu.SemaphoreType.DMA((2,)),
        pltpu.SemaphoreType.DMA((2,)),
        pltpu.SemaphoreType.DMA((2,)),
        pltpu.SemaphoreType.DMA,
        pltpu.VMEM_SHARED((NP, D), jnp.float32),
    ],
)
def _agg_kernel(
    src_hbm, dst_hbm, g_hbm, z_hbm, out_hbm,
    sring, dstage, rows, gsem, isem, ssem, zsem, acc_sh,
):
    cid = lax.axis_index("c")
    sid = lax.axis_index("s")
    w = cid * NS + sid
    ebase = w * EPW
    dcp = pltpu.async_copy(dst_hbm.at[w], dstage, zsem)
    zcp = pltpu.async_copy(
        z_hbm.at[pl.ds(sid * RPT, RPT)], acc_sh.at[pl.ds(sid * RPT, RPT)], zsem
    )

    pltpu.sync_copy(src_hbm.at[pl.ds(ebase, K)], sring.at[0])
    pltpu.async_copy(g_hbm.at[sring.at[0]], rows.at[0], gsem.at[0])
    pltpu.async_copy(src_hbm.at[pl.ds(ebase + K, K)], sring.at[1], isem.at[1])
    dcp.wait()
    zcp.wait()

    def body(b, carry):
        p = lax.rem(b, 2)
        q = 1 - p

        @pl.when(b + 1 < NB)
        def _():
            pltpu.make_async_copy(
                src_hbm.at[pl.ds(ebase, K)], sring.at[q], isem.at[q]
            ).wait()

            @pl.when(b >= 1)
            def _():
                pltpu.make_async_copy(
                    z_hbm.at[pl.ds(0, K)], rows.at[q], ssem.at[q]
                ).wait()

            pltpu.async_copy(g_hbm.at[sring.at[q]], rows.at[q], gsem.at[q])

        pltpu.make_async_copy(
            z_hbm.at[pl.ds(0, K)], rows.at[p], gsem.at[p]
        ).wait()

        @pl.when(b + 2 < NB)
        def _():
            pltpu.async_copy(
                src_hbm.at[pl.ds(ebase + (b + 2) * K, K)],
                sring.at[p],
                isem.at[p],
            )

        pltpu.async_copy(rows.at[p], acc_sh.at[dstage.at[b]], ssem.at[p], add=True)
        return carry

    lax.fori_loop(0, NB, body, 0)

    pltpu.make_async_copy(z_hbm.at[pl.ds(0, K)], rows.at[0], ssem.at[0]).wait()
    pltpu.make_async_copy(z_hbm.at[pl.ds(0, K)], rows.at[1], ssem.at[1]).wait()

    plsc.subcore_barrier()
    pltpu.sync_copy(
        acc_sh.at[pl.ds(sid * RPT, RPT)],
        out_hbm.at[cid, pl.ds(sid * RPT, RPT)],
    )


def _k1_body(x_ref, w_ref, deg_ref, g_ref, dis_ref):
    degp = deg_ref[:, 0, :].astype(jnp.float32)
    deg_col = jnp.dot(
        degp.T, jnp.ones((NW, 1), jnp.float32),
        preferred_element_type=jnp.float32,
    ) + 1.0
    dis = lax.rsqrt(deg_col)
    dis_ref[...] = jnp.broadcast_to(dis, (ROW_BLK, 8))
    h = jnp.dot(x_ref[...], w_ref[...], preferred_element_type=jnp.float32)
    g_ref[...] = dis * h


def _k2_body(acc_ref, g1_ref, dis_ref, w_ref, b_ref, g2_ref):
    dis = dis_ref[:, :1]
    t = dis * (acc_ref[0] + acc_ref[1] + g1_ref[...]) + b_ref[...]
    z = jnp.maximum(t, 0.0)
    h = jnp.dot(z, w_ref[...], preferred_element_type=jnp.float32)
    g2_ref[...] = dis * h


def _k3_body(acc_ref, g2_ref, dis_ref, b_ref, out_ref):
    out_ref[...] = (
        dis_ref[:, :1] * (acc_ref[0] + acc_ref[1] + g2_ref[...]) + b_ref[...]
    )


_row_spec = pl.BlockSpec((ROW_BLK, D), lambda i: (i, 0))
_acc_spec = pl.BlockSpec((NC, ROW_BLK, D), lambda i: (0, i, 0))
_deg_spec = pl.BlockSpec((NW, 8, ROW_BLK), lambda i: (0, 0, i))
_w_spec = pl.BlockSpec((D, D), lambda i: (0, 0))
_b_spec = pl.BlockSpec((1, D), lambda i: (0, 0))
_dis_spec = pl.BlockSpec((ROW_BLK, 8), lambda i: (i, 0))
_dis_struct = jax.ShapeDtypeStruct((NP, 8), jnp.float32)
_out_struct = jax.ShapeDtypeStruct((NP, D), jnp.float32)

_k1 = pl.pallas_call(
    _k1_body,
    grid=(GRID,),
    in_specs=[_row_spec, _w_spec, _deg_spec],
    out_specs=[_row_spec, _dis_spec],
    out_shape=[_out_struct, _dis_struct],
)

_k2 = pl.pallas_call(
    _k2_body,
    grid=(GRID,),
    in_specs=[_acc_spec, _row_spec, _dis_spec, _w_spec, _b_spec],
    out_specs=_row_spec,
    out_shape=_out_struct,
)

_k3 = pl.pallas_call(
    _k3_body,
    grid=(GRID,),
    in_specs=[_acc_spec, _row_spec, _dis_spec, _b_spec],
    out_specs=_row_spec,
    out_shape=_out_struct,
)


def kernel(x, edge_index, W1, b1, W2, b2):
    e32 = edge_index.astype(jnp.int32)
    pad = jnp.arange(EP - E, dtype=jnp.int32)
    src = jnp.concatenate([e32[0], pad % N])
    dst1 = jnp.concatenate([e32[1], N + pad % (NP - N)])
    dst = dst1.reshape(NW, NB, K)
    z128 = jnp.zeros((NP, D), jnp.float32)
    xp = jnp.concatenate([x, jnp.zeros((NP - N, D), x.dtype)])
    b1r = b1.reshape(1, D)
    b2r = b2.reshape(1, D)

    deg = _deg_kernel(dst1)
    g1, dis = _k1(xp, W1, deg)
    acc1 = _agg_kernel(src, dst, g1, z128)
    g2 = _k2(acc1, g1, dis, W2, b1r)
    acc2 = _agg_kernel(src, dst, g2, z128)
    return _k3(acc2, g2, dis, b2r)[:N]
